# baseline (speedup 1.0000x reference)
import functools

import jax
import jax.numpy as jnp
from jax import lax
from jax.experimental import pallas as pl
from jax.experimental.pallas import tpu as pltpu
from jax.experimental.pallas import tpu_sc as plsc

N = 10000
E = 320000
D_IN = 128
D_H = 128
D_OUT = 40

_NC = 2
_NS = 16
_NW = _NC * _NS
_EPT = E // _NW
_CH = 40
_CHK = _EPT // _CH
_G = 2
_S = 5
_DS = 5

_BN = 2000


def _vmesh():
    return plsc.VectorSubcoreMesh(core_axis_name="c", subcore_axis_name="s")


def _make_deg():
    @functools.partial(
        pl.kernel,
        out_type=jax.ShapeDtypeStruct((_NC * N,), jnp.float32),
        mesh=_vmesh(),
        scratch_types=[
            pltpu.VMEM((_CHK, _CH), jnp.int32),
            pltpu.VMEM((_CH,), jnp.float32),
            pltpu.VMEM_SHARED((N,), jnp.float32),
            pltpu.VMEM((N,), jnp.float32),
        ]
        + [pltpu.SemaphoreType.DMA] * _DS,
        compiler_params=pltpu.CompilerParams(use_tc_tiling_on_sc=False),
        name="gcn_deg",
    )
    def deg_kernel(dst3_hbm, out_hbm, didx2, ones_v, acc, buf, *ssem):
        c = lax.axis_index("c")
        s = lax.axis_index("s")
        wid = c * _NS + s

        @pl.when(s == 0)
        def _():
            @pl.loop(0, N // 16)
            def _z(i):
                buf[pl.ds(i * 16, 16)] = jnp.zeros((16,), jnp.float32)

            pltpu.sync_copy(buf, acc)

        for off1 in (0, 16, _CH - 16):
            ones_v[pl.ds(off1, 16)] = jnp.ones((16,), jnp.float32)
        pltpu.sync_copy(dst3_hbm.at[wid], didx2)
        plsc.subcore_barrier()

        def scat(cc, k):
            pltpu.async_copy(ones_v, acc.at[didx2.at[cc]], ssem[k], add=True)

        def scat_wait(cc, k):
            pltpu.make_async_copy(ones_v, acc.at[didx2.at[cc]], ssem[k]).wait()

        for j in range(_DS):
            scat(j, j)

        @pl.loop(0, (_CHK - _DS) // _DS)
        def _main(p):
            for j in range(_DS):
                cc = _DS + p * _DS + j
                scat_wait(cc - _DS, j)
                scat(cc, j)

        for j in range(_DS):
            scat_wait(_CHK - _DS + j, j)

        plsc.subcore_barrier()

        @pl.when(s == 0)
        def _():
            pltpu.sync_copy(acc, buf)
            pltpu.sync_copy(buf, out_hbm.at[pl.ds(c * N, N)])

    return deg_kernel


def _make_agg(D, CH, S, G):
    @functools.partial(
        pl.kernel,
        out_type=jax.ShapeDtypeStruct((_NC * N, D), jnp.float32),
        mesh=_vmesh(),
        scratch_types=[
            pltpu.VMEM((_EPT,), jnp.int32),
            pltpu.VMEM((_EPT // CH, CH), jnp.int32),
            pltpu.VMEM_SHARED((N, D), jnp.float32),
        ]
        + [pltpu.VMEM((CH, D), jnp.float32)] * S
        + [pltpu.SemaphoreType.DMA] * (2 * S),
        compiler_params=pltpu.CompilerParams(use_tc_tiling_on_sc=False),
        name=f"gcn_agg_d{D}",
    )
    def agg_kernel(g_hbm, src_hbm, dst3_hbm, zeros_hbm, out_hbm,
                   sidx, didx2, acc, *bufs):
        CHK = _EPT // CH
        rows = bufs[:S]
        gsem = bufs[S:2 * S]
        ssem = bufs[2 * S:]
        c = lax.axis_index("c")
        s = lax.axis_index("s")
        wid = c * _NS + s

        @pl.when(s < _NS - 1)
        def _():
            pltpu.sync_copy(zeros_hbm.at[pl.ds(s * 640, 640)],
                            acc.at[pl.ds(s * 640, 640)])

        @pl.when(s == _NS - 1)
        def _():
            pltpu.sync_copy(zeros_hbm.at[pl.ds(9600, 400)],
                            acc.at[pl.ds(9600, 400)])

        pltpu.sync_copy(src_hbm.at[pl.ds(wid * _EPT, _EPT)], sidx)
        pltpu.sync_copy(dst3_hbm.at[wid], didx2)
        plsc.subcore_barrier()

        def gath(cc, k):
            pltpu.async_copy(g_hbm.at[sidx.at[pl.ds(cc * CH, CH)]],
                             rows[k], gsem[k])

        def gath_wait(cc, k):
            pltpu.make_async_copy(g_hbm.at[sidx.at[pl.ds(cc * CH, CH)]],
                                  rows[k], gsem[k]).wait()

        def scat(cc, k):
            pltpu.async_copy(rows[k], acc.at[didx2.at[cc]], ssem[k], add=True)

        def scat_wait(cc, k):
            pltpu.make_async_copy(rows[k], acc.at[didx2.at[cc]],
                                  ssem[k]).wait()

        HEAD = S - G
        R = (CHK - HEAD - G) // S
        TAIL = CHK - HEAD - G - R * S
        for j in range(G):
            gath(j, j)
        for cc in range(HEAD):
            gath_wait(cc, cc % S)
            scat(cc, cc % S)
            gath(cc + G, (cc + G) % S)

        @pl.loop(0, R)
        def _main(p):
            for j in range(S):
                cc = HEAD + p * S + j
                k = (HEAD + j) % S
                k2 = (HEAD + j + G) % S
                gath_wait(cc, k)
                scat(cc, k)
                scat_wait(cc - (S - G), k2)
                gath(cc + G, k2)

        for j in range(TAIL):
            cc = HEAD + R * S + j
            gath_wait(cc, cc % S)
            scat(cc, cc % S)
            scat_wait(cc - (S - G), (cc + G) % S)
            gath(cc + G, (cc + G) % S)
        for j in range(G):
            cc = CHK - G + j
            gath_wait(cc, cc % S)
            scat(cc, cc % S)
        for j in range(S):
            cc = CHK - S + j
            scat_wait(cc, cc % S)

        plsc.subcore_barrier()

        @pl.when(s < _NS - 1)
        def _():
            pltpu.sync_copy(acc.at[pl.ds(s * 640, 640)],
                            out_hbm.at[pl.ds(c * N + s * 640, 640)])

        @pl.when(s == _NS - 1)
        def _():
            pltpu.sync_copy(acc.at[pl.ds(9600, 400)],
                            out_hbm.at[pl.ds(c * N + 9600, 400)])

    return agg_kernel


_deg_kernel = _make_deg()
_agg_h = _make_agg(D_H, 16, 14, 11)
_agg_o = _make_agg(D_OUT, 80, 9, 6)


def _tc_first(x, W1, deg2):
    def body(x_ref, w_ref, d0_ref, d1_ref, g_ref, dinv_ref):
        deg = d0_ref[...] + d1_ref[...] + 1.0
        dinv = lax.rsqrt(deg)
        h = lax.dot_general(x_ref[...], w_ref[...], (((1,), (1,)), ((), ())),
                            preferred_element_type=jnp.float32)
        g_ref[...] = h * dinv
        dinv_ref[...] = dinv

    nb = N // _BN
    return pl.pallas_call(
        body,
        grid=(nb,),
        in_specs=[
            pl.BlockSpec((_BN, D_IN), lambda i: (i, 0)),
            pl.BlockSpec((D_H, D_IN), lambda i: (0, 0)),
            pl.BlockSpec((_BN, 1), lambda i: (i, 0)),
            pl.BlockSpec((_BN, 1), lambda i: (i + nb, 0)),
        ],
        out_specs=[
            pl.BlockSpec((_BN, D_H), lambda i: (i, 0)),
            pl.BlockSpec((_BN, 1), lambda i: (i, 0)),
        ],
        out_shape=[
            jax.ShapeDtypeStruct((N, D_H), jnp.float32),
            jax.ShapeDtypeStruct((N, 1), jnp.float32),
        ],
        name="gcn_tc_first",
    )(x, W1, deg2, deg2)


def _tc_mid(a_flat, g_prev, dinv, b_prev, W_next, d_next):
    d_prev = g_prev.shape[1]

    def body(a0_ref, a1_ref, g_ref, dinv_ref, b_ref, w_ref, o_ref):
        dinv = dinv_ref[...]
        sm = (a0_ref[...] + a1_ref[...] + g_ref[...]) * dinv + b_ref[...]
        z = jnp.maximum(sm, 0.0)
        h = lax.dot_general(z, w_ref[...], (((1,), (1,)), ((), ())),
                            preferred_element_type=jnp.float32)
        o_ref[...] = h * dinv

    nb = N // _BN
    return pl.pallas_call(
        body,
        grid=(nb,),
        in_specs=[
            pl.BlockSpec((_BN, d_prev), lambda i: (i, 0)),
            pl.BlockSpec((_BN, d_prev), lambda i: (i + nb, 0)),
            pl.BlockSpec((_BN, d_prev), lambda i: (i, 0)),
            pl.BlockSpec((_BN, 1), lambda i: (i, 0)),
            pl.BlockSpec((1, d_prev), lambda i: (0, 0)),
            pl.BlockSpec((d_next, d_prev), lambda i: (0, 0)),
        ],
        out_specs=pl.BlockSpec((_BN, d_next), lambda i: (i, 0)),
        out_shape=jax.ShapeDtypeStruct((N, d_next), jnp.float32),
        name=f"gcn_tc_mid_{d_next}",
    )(a_flat, a_flat, g_prev, dinv, b_prev.reshape(1, d_prev), W_next)


def _tc_last(a_flat, g_prev, dinv, b):
    d = g_prev.shape[1]

    def body(a0_ref, a1_ref, g_ref, dinv_ref, b_ref, o_ref):
        o_ref[...] = ((a0_ref[...] + a1_ref[...] + g_ref[...]) * dinv_ref[...]
                      + b_ref[...])

    nb = N // _BN
    return pl.pallas_call(
        body,
        grid=(nb,),
        in_specs=[
            pl.BlockSpec((_BN, d), lambda i: (i, 0)),
            pl.BlockSpec((_BN, d), lambda i: (i + nb, 0)),
            pl.BlockSpec((_BN, d), lambda i: (i, 0)),
            pl.BlockSpec((_BN, 1), lambda i: (i, 0)),
            pl.BlockSpec((1, d), lambda i: (0, 0)),
        ],
        out_specs=pl.BlockSpec((_BN, d), lambda i: (i, 0)),
        out_shape=jax.ShapeDtypeStruct((N, d), jnp.float32),
        name="gcn_tc_last",
    )(a_flat, a_flat, g_prev, dinv, b.reshape(1, d))


def kernel(x, edge_index, W1, b1, W2, b2, W3, b3):
    ei = edge_index.astype(jnp.int32)
    src = ei[0]
    dst3_16 = ei[1].reshape(_NW, _EPT // 16, 16)
    dst3_40 = ei[1].reshape(_NW, _EPT // 40, 40)
    dst3_80 = ei[1].reshape(_NW, _EPT // 80, 80)
    zeros_nd = jnp.zeros((N, D_H), jnp.float32)
    zeros_no = jnp.zeros((N, D_OUT), jnp.float32)

    deg2 = _deg_kernel(dst3_40).reshape(_NC * N, 1)

    g1, dinv = _tc_first(x, W1, deg2)
    a1 = _agg_h(g1, src, dst3_16, zeros_nd)
    g2 = _tc_mid(a1, g1, dinv, b1, W2, D_H)
    a2 = _agg_h(g2, src, dst3_16, zeros_nd)
    g3 = _tc_mid(a2, g2, dinv, b2, W3, D_OUT)
    a3 = _agg_o(g3, src, dst3_80, zeros_no)
    return _tc_last(a3, g3, dinv, b3)


# speedup vs baseline: 1.0398x; 1.0398x over previous
import functools

import jax
import jax.numpy as jnp
from jax import lax
from jax.experimental import pallas as pl
from jax.experimental.pallas import tpu as pltpu
from jax.experimental.pallas import tpu_sc as plsc

N = 10000
E = 320000
D_IN = 128
D_H = 128
D_OUT = 40

_NC = 2
_NS = 16
_NW = _NC * _NS
_EPT = E // _NW
_CH = 40
_CHK = _EPT // _CH
_G = 2
_S = 5
_DS = 5

_BN = 2000


def _vmesh():
    return plsc.VectorSubcoreMesh(core_axis_name="c", subcore_axis_name="s")


def _make_deg():
    @functools.partial(
        pl.kernel,
        out_type=jax.ShapeDtypeStruct((_NC * N,), jnp.float32),
        mesh=_vmesh(),
        scratch_types=[
            pltpu.VMEM((_CHK, _CH), jnp.int32),
            pltpu.VMEM((_CH,), jnp.float32),
            pltpu.VMEM_SHARED((N,), jnp.float32),
            pltpu.VMEM((N,), jnp.float32),
        ]
        + [pltpu.SemaphoreType.DMA] * _DS,
        compiler_params=pltpu.CompilerParams(use_tc_tiling_on_sc=False),
        name="gcn_deg",
    )
    def deg_kernel(dst3_hbm, out_hbm, didx2, ones_v, acc, buf, *ssem):
        c = lax.axis_index("c")
        s = lax.axis_index("s")
        wid = c * _NS + s

        @pl.when(s == 0)
        def _():
            @pl.loop(0, N // 16)
            def _z(i):
                buf[pl.ds(i * 16, 16)] = jnp.zeros((16,), jnp.float32)

            pltpu.sync_copy(buf, acc)

        for off1 in (0, 16, _CH - 16):
            ones_v[pl.ds(off1, 16)] = jnp.ones((16,), jnp.float32)
        pltpu.sync_copy(dst3_hbm.at[wid], didx2)
        plsc.subcore_barrier()

        def scat(cc, k):
            pltpu.async_copy(ones_v, acc.at[didx2.at[cc]], ssem[k], add=True)

        def scat_wait(cc, k):
            pltpu.make_async_copy(ones_v, acc.at[didx2.at[cc]], ssem[k]).wait()

        for j in range(_DS):
            scat(j, j)

        @pl.loop(0, (_CHK - _DS) // _DS)
        def _main(p):
            for j in range(_DS):
                cc = _DS + p * _DS + j
                scat_wait(cc - _DS, j)
                scat(cc, j)

        for j in range(_DS):
            scat_wait(_CHK - _DS + j, j)

        plsc.subcore_barrier()

        @pl.when(s == 0)
        def _():
            pltpu.sync_copy(acc, buf)
            pltpu.sync_copy(buf, out_hbm.at[pl.ds(c * N, N)])

    return deg_kernel


def _make_agg(D, CH, S, G):
    @functools.partial(
        pl.kernel,
        out_type=jax.ShapeDtypeStruct((_NC * N, D), jnp.float32),
        mesh=_vmesh(),
        scratch_types=[
            pltpu.VMEM((_EPT,), jnp.int32),
            pltpu.VMEM((_EPT // CH, CH), jnp.int32),
            pltpu.VMEM_SHARED((N, D), jnp.float32),
        ]
        + [pltpu.VMEM((CH, D), jnp.float32)] * S
        + [pltpu.SemaphoreType.DMA] * (2 * S),
        compiler_params=pltpu.CompilerParams(use_tc_tiling_on_sc=False),
        name=f"gcn_agg_d{D}",
    )
    def agg_kernel(g_hbm, src_hbm, dst3_hbm, zeros_hbm, out_hbm,
                   sidx, didx2, acc, *bufs):
        CHK = _EPT // CH
        rows = bufs[:S]
        gsem = bufs[S:2 * S]
        ssem = bufs[2 * S:]
        c = lax.axis_index("c")
        s = lax.axis_index("s")
        wid = c * _NS + s

        @pl.when(s < _NS - 1)
        def _():
            pltpu.sync_copy(zeros_hbm.at[pl.ds(s * 640, 640)],
                            acc.at[pl.ds(s * 640, 640)])

        @pl.when(s == _NS - 1)
        def _():
            pltpu.sync_copy(zeros_hbm.at[pl.ds(9600, 400)],
                            acc.at[pl.ds(9600, 400)])

        pltpu.sync_copy(src_hbm.at[pl.ds(wid * _EPT, _EPT)], sidx)
        pltpu.sync_copy(dst3_hbm.at[wid], didx2)
        plsc.subcore_barrier()

        def gath(cc, k):
            pltpu.async_copy(g_hbm.at[sidx.at[pl.ds(cc * CH, CH)]],
                             rows[k], gsem[k])

        def gath_wait(cc, k):
            pltpu.make_async_copy(g_hbm.at[sidx.at[pl.ds(cc * CH, CH)]],
                                  rows[k], gsem[k]).wait()

        def scat(cc, k):
            pltpu.async_copy(rows[k], acc.at[didx2.at[cc]], ssem[k], add=True)

        def scat_wait(cc, k):
            pltpu.make_async_copy(rows[k], acc.at[didx2.at[cc]],
                                  ssem[k]).wait()

        HEAD = S - G
        R = (CHK - HEAD - G) // S
        TAIL = CHK - HEAD - G - R * S
        for j in range(G):
            gath(j, j)
        for cc in range(HEAD):
            gath_wait(cc, cc % S)
            scat(cc, cc % S)
            gath(cc + G, (cc + G) % S)

        @pl.loop(0, R)
        def _main(p):
            for j in range(S):
                cc = HEAD + p * S + j
                k = (HEAD + j) % S
                k2 = (HEAD + j + G) % S
                gath_wait(cc, k)
                scat(cc, k)
                scat_wait(cc - (S - G), k2)
                gath(cc + G, k2)

        for j in range(TAIL):
            cc = HEAD + R * S + j
            gath_wait(cc, cc % S)
            scat(cc, cc % S)
            scat_wait(cc - (S - G), (cc + G) % S)
            gath(cc + G, (cc + G) % S)
        for j in range(G):
            cc = CHK - G + j
            gath_wait(cc, cc % S)
            scat(cc, cc % S)
        for j in range(S):
            cc = CHK - S + j
            scat_wait(cc, cc % S)

        plsc.subcore_barrier()

        @pl.when(s < _NS - 1)
        def _():
            pltpu.sync_copy(acc.at[pl.ds(s * 640, 640)],
                            out_hbm.at[pl.ds(c * N + s * 640, 640)])

        @pl.when(s == _NS - 1)
        def _():
            pltpu.sync_copy(acc.at[pl.ds(9600, 400)],
                            out_hbm.at[pl.ds(c * N + 9600, 400)])

    return agg_kernel


_deg_kernel = _make_deg()
_agg_h = _make_agg(D_H, 16, 14, 9)
_agg_o = _make_agg(D_OUT, 80, 12, 7)


def _tc_first(x, W1, deg2):
    def body(x_ref, w_ref, d0_ref, d1_ref, g_ref, dinv_ref):
        deg = d0_ref[...] + d1_ref[...] + 1.0
        dinv = lax.rsqrt(deg)
        h = lax.dot_general(x_ref[...], w_ref[...], (((1,), (1,)), ((), ())),
                            preferred_element_type=jnp.float32)
        g_ref[...] = h * dinv
        dinv_ref[...] = dinv

    nb = N // _BN
    return pl.pallas_call(
        body,
        grid=(nb,),
        in_specs=[
            pl.BlockSpec((_BN, D_IN), lambda i: (i, 0)),
            pl.BlockSpec((D_H, D_IN), lambda i: (0, 0)),
            pl.BlockSpec((_BN, 1), lambda i: (i, 0)),
            pl.BlockSpec((_BN, 1), lambda i: (i + nb, 0)),
        ],
        out_specs=[
            pl.BlockSpec((_BN, D_H), lambda i: (i, 0)),
            pl.BlockSpec((_BN, 1), lambda i: (i, 0)),
        ],
        out_shape=[
            jax.ShapeDtypeStruct((N, D_H), jnp.float32),
            jax.ShapeDtypeStruct((N, 1), jnp.float32),
        ],
        name="gcn_tc_first",
    )(x, W1, deg2, deg2)


def _tc_mid(a_flat, g_prev, dinv, b_prev, W_next, d_next):
    d_prev = g_prev.shape[1]

    def body(a0_ref, a1_ref, g_ref, dinv_ref, b_ref, w_ref, o_ref):
        dinv = dinv_ref[...]
        sm = (a0_ref[...] + a1_ref[...] + g_ref[...]) * dinv + b_ref[...]
        z = jnp.maximum(sm, 0.0)
        h = lax.dot_general(z, w_ref[...], (((1,), (1,)), ((), ())),
                            preferred_element_type=jnp.float32)
        o_ref[...] = h * dinv

    nb = N // _BN
    return pl.pallas_call(
        body,
        grid=(nb,),
        in_specs=[
            pl.BlockSpec((_BN, d_prev), lambda i: (i, 0)),
            pl.BlockSpec((_BN, d_prev), lambda i: (i + nb, 0)),
            pl.BlockSpec((_BN, d_prev), lambda i: (i, 0)),
            pl.BlockSpec((_BN, 1), lambda i: (i, 0)),
            pl.BlockSpec((1, d_prev), lambda i: (0, 0)),
            pl.BlockSpec((d_next, d_prev), lambda i: (0, 0)),
        ],
        out_specs=pl.BlockSpec((_BN, d_next), lambda i: (i, 0)),
        out_shape=jax.ShapeDtypeStruct((N, d_next), jnp.float32),
        name=f"gcn_tc_mid_{d_next}",
    )(a_flat, a_flat, g_prev, dinv, b_prev.reshape(1, d_prev), W_next)


def _tc_last(a_flat, g_prev, dinv, b):
    d = g_prev.shape[1]

    def body(a0_ref, a1_ref, g_ref, dinv_ref, b_ref, o_ref):
        o_ref[...] = ((a0_ref[...] + a1_ref[...] + g_ref[...]) * dinv_ref[...]
                      + b_ref[...])

    nb = N // _BN
    return pl.pallas_call(
        body,
        grid=(nb,),
        in_specs=[
            pl.BlockSpec((_BN, d), lambda i: (i, 0)),
            pl.BlockSpec((_BN, d), lambda i: (i + nb, 0)),
            pl.BlockSpec((_BN, d), lambda i: (i, 0)),
            pl.BlockSpec((_BN, 1), lambda i: (i, 0)),
            pl.BlockSpec((1, d), lambda i: (0, 0)),
        ],
        out_specs=pl.BlockSpec((_BN, d), lambda i: (i, 0)),
        out_shape=jax.ShapeDtypeStruct((N, d), jnp.float32),
        name="gcn_tc_last",
    )(a_flat, a_flat, g_prev, dinv, b.reshape(1, d))


def kernel(x, edge_index, W1, b1, W2, b2, W3, b3):
    ei = edge_index.astype(jnp.int32)
    src = ei[0]
    dst3_16 = ei[1].reshape(_NW, _EPT // 16, 16)
    dst3_40 = ei[1].reshape(_NW, _EPT // 40, 40)
    dst3_80 = ei[1].reshape(_NW, _EPT // 80, 80)
    zeros_nd = jnp.zeros((N, D_H), jnp.float32)
    zeros_no = jnp.zeros((N, D_OUT), jnp.float32)

    deg2 = _deg_kernel(dst3_40).reshape(_NC * N, 1)

    g1, dinv = _tc_first(x, W1, deg2)
    a1 = _agg_h(g1, src, dst3_16, zeros_nd)
    g2 = _tc_mid(a1, g1, dinv, b1, W2, D_H)
    a2 = _agg_h(g2, src, dst3_16, zeros_nd)
    g3 = _tc_mid(a2, g2, dinv, b2, W3, D_OUT)
    a3 = _agg_o(g3, src, dst3_80, zeros_no)
    return _tc_last(a3, g3, dinv, b3)


# speedup vs baseline: 1.0430x; 1.0030x over previous
import functools

import jax
import jax.numpy as jnp
from jax import lax
from jax.experimental import pallas as pl
from jax.experimental.pallas import tpu as pltpu
from jax.experimental.pallas import tpu_sc as plsc

N = 10000
E = 320000
D_IN = 128
D_H = 128
D_OUT = 40

_NC = 2
_NS = 16
_NW = _NC * _NS
_EPT = E // _NW
_CH = 40
_CHK = _EPT // _CH
_G = 2
_S = 5
_DS = 5
_DCH = 80
_DCHK = _EPT // _DCH

_BN = 2000


def _vmesh():
    return plsc.VectorSubcoreMesh(core_axis_name="c", subcore_axis_name="s")


def _make_deg():
    @functools.partial(
        pl.kernel,
        out_type=jax.ShapeDtypeStruct((_NC * N,), jnp.float32),
        mesh=_vmesh(),
        scratch_types=[
            pltpu.VMEM((_EPT,), jnp.int32),
            pltpu.VMEM((_DCH,), jnp.float32),
            pltpu.VMEM_SHARED((N,), jnp.float32),
            pltpu.VMEM((N,), jnp.float32),
        ]
        + [pltpu.SemaphoreType.DMA] * _DS,
        compiler_params=pltpu.CompilerParams(use_tc_tiling_on_sc=False),
        name="gcn_deg",
    )
    def deg_kernel(dst_hbm, out_hbm, didx, ones_v, acc, buf, *ssem):
        c = lax.axis_index("c")
        s = lax.axis_index("s")
        wid = c * _NS + s

        @pl.when(s == 0)
        def _():
            @pl.loop(0, N // 16)
            def _z(i):
                buf[pl.ds(i * 16, 16)] = jnp.zeros((16,), jnp.float32)

            pltpu.sync_copy(buf, acc)

        for off1 in range(0, _DCH, 16):
            ones_v[pl.ds(off1, 16)] = jnp.ones((16,), jnp.float32)
        pltpu.sync_copy(dst_hbm.at[pl.ds(wid * _EPT, _EPT)], didx)
        plsc.subcore_barrier()

        def scat(cc, k):
            pltpu.async_copy(ones_v, acc.at[didx.at[pl.ds(cc * _DCH, _DCH)]],
                             ssem[k], add=True)

        def scat_wait(cc, k):
            pltpu.make_async_copy(ones_v,
                                  acc.at[didx.at[pl.ds(cc * _DCH, _DCH)]],
                                  ssem[k]).wait()

        for j in range(_DS):
            scat(j, j)

        @pl.loop(0, (_DCHK - _DS) // _DS)
        def _main(p):
            for j in range(_DS):
                cc = _DS + p * _DS + j
                scat_wait(cc - _DS, j)
                scat(cc, j)

        for j in range(_DS):
            scat_wait(_DCHK - _DS + j, j)

        plsc.subcore_barrier()

        @pl.when(s == 0)
        def _():
            pltpu.sync_copy(acc, buf)
            pltpu.sync_copy(buf, out_hbm.at[pl.ds(c * N, N)])

    return deg_kernel


def _make_agg(D, CH, S, G):
    @functools.partial(
        pl.kernel,
        out_type=jax.ShapeDtypeStruct((_NC * N, D), jnp.float32),
        mesh=_vmesh(),
        scratch_types=[
            pltpu.VMEM((_EPT,), jnp.int32),
            pltpu.VMEM((_EPT,), jnp.int32),
            pltpu.VMEM_SHARED((N, D), jnp.float32),
        ]
        + [pltpu.VMEM((CH, D), jnp.float32)] * S
        + [pltpu.SemaphoreType.DMA] * (2 * S),
        compiler_params=pltpu.CompilerParams(use_tc_tiling_on_sc=False),
        name=f"gcn_agg_d{D}",
    )
    def agg_kernel(g_hbm, src_hbm, dst_hbm, zeros_hbm, out_hbm,
                   sidx, didx, acc, *bufs):
        CHK = _EPT // CH
        rows = bufs[:S]
        gsem = bufs[S:2 * S]
        ssem = bufs[2 * S:]
        c = lax.axis_index("c")
        s = lax.axis_index("s")
        wid = c * _NS + s

        @pl.when(s < _NS - 1)
        def _():
            pltpu.sync_copy(zeros_hbm.at[pl.ds(s * 640, 640)],
                            acc.at[pl.ds(s * 640, 640)])

        @pl.when(s == _NS - 1)
        def _():
            pltpu.sync_copy(zeros_hbm.at[pl.ds(9600, 400)],
                            acc.at[pl.ds(9600, 400)])

        pltpu.sync_copy(src_hbm.at[pl.ds(wid * _EPT, _EPT)], sidx)
        pltpu.sync_copy(dst_hbm.at[pl.ds(wid * _EPT, _EPT)], didx)
        plsc.subcore_barrier()

        def gath(cc, k):
            pltpu.async_copy(g_hbm.at[sidx.at[pl.ds(cc * CH, CH)]],
                             rows[k], gsem[k])

        def gath_wait(cc, k):
            pltpu.make_async_copy(g_hbm.at[sidx.at[pl.ds(cc * CH, CH)]],
                                  rows[k], gsem[k]).wait()

        def scat(cc, k):
            pltpu.async_copy(rows[k], acc.at[didx.at[pl.ds(cc * CH, CH)]],
                             ssem[k], add=True)

        def scat_wait(cc, k):
            pltpu.make_async_copy(rows[k],
                                  acc.at[didx.at[pl.ds(cc * CH, CH)]],
                                  ssem[k]).wait()

        HEAD = S - G
        R = (CHK - HEAD - G) // S
        TAIL = CHK - HEAD - G - R * S
        for j in range(G):
            gath(j, j)
        for cc in range(HEAD):
            gath_wait(cc, cc % S)
            scat(cc, cc % S)
            gath(cc + G, (cc + G) % S)

        @pl.loop(0, R)
        def _main(p):
            for j in range(S):
                cc = HEAD + p * S + j
                k = (HEAD + j) % S
                k2 = (HEAD + j + G) % S
                gath_wait(cc, k)
                scat(cc, k)
                scat_wait(cc - (S - G), k2)
                gath(cc + G, k2)

        for j in range(TAIL):
            cc = HEAD + R * S + j
            gath_wait(cc, cc % S)
            scat(cc, cc % S)
            scat_wait(cc - (S - G), (cc + G) % S)
            gath(cc + G, (cc + G) % S)
        for j in range(G):
            cc = CHK - G + j
            gath_wait(cc, cc % S)
            scat(cc, cc % S)
        for j in range(S):
            cc = CHK - S + j
            scat_wait(cc, cc % S)

        plsc.subcore_barrier()

        @pl.when(s < _NS - 1)
        def _():
            pltpu.sync_copy(acc.at[pl.ds(s * 640, 640)],
                            out_hbm.at[pl.ds(c * N + s * 640, 640)])

        @pl.when(s == _NS - 1)
        def _():
            pltpu.sync_copy(acc.at[pl.ds(9600, 400)],
                            out_hbm.at[pl.ds(c * N + 9600, 400)])

    return agg_kernel


_deg_kernel = _make_deg()
_agg_h = _make_agg(D_H, 16, 14, 9)
_agg_o = _make_agg(D_OUT, 80, 12, 7)


def _tc_first(x, W1, deg2):
    def body(x_ref, w_ref, d0_ref, d1_ref, g_ref, dinv_ref):
        deg = d0_ref[...] + d1_ref[...] + 1.0
        dinv = lax.rsqrt(deg)
        h = lax.dot_general(x_ref[...], w_ref[...], (((1,), (1,)), ((), ())),
                            preferred_element_type=jnp.float32)
        g_ref[...] = h * dinv
        dinv_ref[...] = dinv

    nb = N // _BN
    return pl.pallas_call(
        body,
        grid=(nb,),
        in_specs=[
            pl.BlockSpec((_BN, D_IN), lambda i: (i, 0)),
            pl.BlockSpec((D_H, D_IN), lambda i: (0, 0)),
            pl.BlockSpec((_BN, 1), lambda i: (i, 0)),
            pl.BlockSpec((_BN, 1), lambda i: (i + nb, 0)),
        ],
        out_specs=[
            pl.BlockSpec((_BN, D_H), lambda i: (i, 0)),
            pl.BlockSpec((_BN, 1), lambda i: (i, 0)),
        ],
        out_shape=[
            jax.ShapeDtypeStruct((N, D_H), jnp.float32),
            jax.ShapeDtypeStruct((N, 1), jnp.float32),
        ],
        name="gcn_tc_first",
    )(x, W1, deg2, deg2)


def _tc_mid(a_flat, g_prev, dinv, b_prev, W_next, d_next):
    d_prev = g_prev.shape[1]

    def body(a0_ref, a1_ref, g_ref, dinv_ref, b_ref, w_ref, o_ref):
        dinv = dinv_ref[...]
        sm = (a0_ref[...] + a1_ref[...] + g_ref[...]) * dinv + b_ref[...]
        z = jnp.maximum(sm, 0.0)
        h = lax.dot_general(z, w_ref[...], (((1,), (1,)), ((), ())),
                            preferred_element_type=jnp.float32)
        o_ref[...] = h * dinv

    nb = N // _BN
    return pl.pallas_call(
        body,
        grid=(nb,),
        in_specs=[
            pl.BlockSpec((_BN, d_prev), lambda i: (i, 0)),
            pl.BlockSpec((_BN, d_prev), lambda i: (i + nb, 0)),
            pl.BlockSpec((_BN, d_prev), lambda i: (i, 0)),
            pl.BlockSpec((_BN, 1), lambda i: (i, 0)),
            pl.BlockSpec((1, d_prev), lambda i: (0, 0)),
            pl.BlockSpec((d_next, d_prev), lambda i: (0, 0)),
        ],
        out_specs=pl.BlockSpec((_BN, d_next), lambda i: (i, 0)),
        out_shape=jax.ShapeDtypeStruct((N, d_next), jnp.float32),
        name=f"gcn_tc_mid_{d_next}",
    )(a_flat, a_flat, g_prev, dinv, b_prev.reshape(1, d_prev), W_next)


def _tc_last(a_flat, g_prev, dinv, b):
    d = g_prev.shape[1]

    def body(a0_ref, a1_ref, g_ref, dinv_ref, b_ref, o_ref):
        o_ref[...] = ((a0_ref[...] + a1_ref[...] + g_ref[...]) * dinv_ref[...]
                      + b_ref[...])

    nb = N // _BN
    return pl.pallas_call(
        body,
        grid=(nb,),
        in_specs=[
            pl.BlockSpec((_BN, d), lambda i: (i, 0)),
            pl.BlockSpec((_BN, d), lambda i: (i + nb, 0)),
            pl.BlockSpec((_BN, d), lambda i: (i, 0)),
            pl.BlockSpec((_BN, 1), lambda i: (i, 0)),
            pl.BlockSpec((1, d), lambda i: (0, 0)),
        ],
        out_specs=pl.BlockSpec((_BN, d), lambda i: (i, 0)),
        out_shape=jax.ShapeDtypeStruct((N, d), jnp.float32),
        name="gcn_tc_last",
    )(a_flat, a_flat, g_prev, dinv, b.reshape(1, d))


def kernel(x, edge_index, W1, b1, W2, b2, W3, b3):
    ei = edge_index.astype(jnp.int32)
    src = ei[0]
    dst = ei[1]
    zeros_nd = jnp.zeros((N, D_H), jnp.float32)
    zeros_no = jnp.zeros((N, D_OUT), jnp.float32)

    deg2 = _deg_kernel(dst).reshape(_NC * N, 1)

    g1, dinv = _tc_first(x, W1, deg2)
    a1 = _agg_h(g1, src, dst, zeros_nd)
    g2 = _tc_mid(a1, g1, dinv, b1, W2, D_H)
    a2 = _agg_h(g2, src, dst, zeros_nd)
    g3 = _tc_mid(a2, g2, dinv, b2, W3, D_OUT)
    a3 = _agg_o(g3, src, dst, zeros_no)
    return _tc_last(a3, g3, dinv, b3)
